# baseline (speedup 1.0000x reference)
import jax
import jax.numpy as jnp
from jax.experimental import pallas as pl
from jax.experimental.pallas import tpu as pltpu

BT = 2048
NBUF = 4


def _router_block(x_hbm, w_ref, b_ref, logits_ref, probs_ref, buf, sems):
    i = pl.program_id(0)
    nsteps = pl.num_programs(0)

    def copy_in(step, slot):
        pltpu.make_async_copy(
            x_hbm.at[pl.ds(step * BT, BT), :],
            buf.at[slot],
            sems.at[slot],
        ).start()

    @pl.when(i == 0)
    def _prologue():
        for k in range(NBUF - 1):
            copy_in(k, k)

    nxt = i + NBUF - 1

    @pl.when(nxt < nsteps)
    def _issue_ahead():
        copy_in(nxt, jax.lax.rem(nxt, NBUF))

    slot = jax.lax.rem(i, NBUF)
    pltpu.make_async_copy(
        x_hbm.at[pl.ds(i * BT, BT), :],
        buf.at[slot],
        sems.at[slot],
    ).wait()

    logits = jnp.dot(buf[slot], w_ref[...], preferred_element_type=jnp.float32)
    logits = logits + b_ref[...]
    logits_ref[...] = logits
    m = jnp.max(logits, axis=-1, keepdims=True)
    e = jnp.exp(logits - m)
    probs_ref[...] = e / jnp.sum(e, axis=-1, keepdims=True)


def kernel(x, W, b):
    tokens, d = x.shape
    n_adapters = W.shape[1]
    b2 = b.reshape(1, n_adapters)
    out_shape = jax.ShapeDtypeStruct((tokens, n_adapters), jnp.float32)
    logits, probs = pl.pallas_call(
        _router_block,
        grid=(tokens // BT,),
        in_specs=[
            pl.BlockSpec(memory_space=pltpu.HBM),
            pl.BlockSpec((d, n_adapters), lambda i: (0, 0)),
            pl.BlockSpec((1, n_adapters), lambda i: (0, 0)),
        ],
        out_specs=[
            pl.BlockSpec((BT, n_adapters), lambda i: (i, 0)),
            pl.BlockSpec((BT, n_adapters), lambda i: (i, 0)),
        ],
        out_shape=[out_shape, out_shape],
        scratch_shapes=[
            pltpu.VMEM((NBUF, BT, d), jnp.float32),
            pltpu.SemaphoreType.DMA((NBUF,)),
        ],
        compiler_params=pltpu.CompilerParams(
            dimension_semantics=(pltpu.ARBITRARY,),
            vmem_limit_bytes=100 * 1024 * 1024,
        ),
    )(x, W, b2)
    return (logits, probs)


# speedup vs baseline: 1.0025x; 1.0025x over previous
import jax
import jax.numpy as jnp
from jax.experimental import pallas as pl
from jax.experimental.pallas import tpu as pltpu

BT = 2048
NBUF = 4
SUBS = 4
SUBT = BT // SUBS


def _router_block(x_hbm, w_ref, b_ref, logits_ref, probs_ref, buf, sems):
    i = pl.program_id(0)
    nsteps = pl.num_programs(0)

    def sub_copy(step, slot, j):
        return pltpu.make_async_copy(
            x_hbm.at[pl.ds(step * BT + j * SUBT, SUBT), :],
            buf.at[slot, pl.ds(j * SUBT, SUBT), :],
            sems.at[slot],
        )

    def copy_in(step, slot):
        for j in range(SUBS):
            sub_copy(step, slot, j).start()

    @pl.when(i == 0)
    def _prologue():
        for k in range(NBUF - 1):
            copy_in(k, k)

    nxt = i + NBUF - 1

    @pl.when(nxt < nsteps)
    def _issue_ahead():
        copy_in(nxt, jax.lax.rem(nxt, NBUF))

    slot = jax.lax.rem(i, NBUF)
    for j in range(SUBS):
        sub_copy(i, slot, j).wait()

    logits = jnp.dot(buf[slot], w_ref[...], preferred_element_type=jnp.float32)
    logits = logits + b_ref[...]
    logits_ref[...] = logits
    m = jnp.max(logits, axis=-1, keepdims=True)
    e = jnp.exp(logits - m)
    probs_ref[...] = e / jnp.sum(e, axis=-1, keepdims=True)


def kernel(x, W, b):
    tokens, d = x.shape
    n_adapters = W.shape[1]
    b2 = b.reshape(1, n_adapters)
    out_shape = jax.ShapeDtypeStruct((tokens, n_adapters), jnp.float32)
    logits, probs = pl.pallas_call(
        _router_block,
        grid=(tokens // BT,),
        in_specs=[
            pl.BlockSpec(memory_space=pltpu.HBM),
            pl.BlockSpec((d, n_adapters), lambda i: (0, 0)),
            pl.BlockSpec((1, n_adapters), lambda i: (0, 0)),
        ],
        out_specs=[
            pl.BlockSpec((BT, n_adapters), lambda i: (i, 0)),
            pl.BlockSpec((BT, n_adapters), lambda i: (i, 0)),
        ],
        out_shape=[out_shape, out_shape],
        scratch_shapes=[
            pltpu.VMEM((NBUF, BT, d), jnp.float32),
            pltpu.SemaphoreType.DMA((NBUF,)),
        ],
        compiler_params=pltpu.CompilerParams(
            dimension_semantics=(pltpu.ARBITRARY,),
            vmem_limit_bytes=100 * 1024 * 1024,
        ),
    )(x, W, b2)
    return (logits, probs)
